# baseline (speedup 1.0000x reference)
import functools

import jax
import jax.numpy as jnp
from jax import lax
from jax.experimental import pallas as pl
from jax.experimental.pallas import tpu as pltpu
from jax.experimental.pallas import tpu_sc as plsc

B, L, D = 4096, 50, 32
NC, NS, LANES = 2, 16, 16
NW = NC * NS
ROWS_W = B // NW
LOOK_W = ROWS_W * L
LC = 5
CHUNK = LC * ROWS_W
N_CHUNK = L // LC
VPR = ROWS_W // LANES


def _sc_body(tab_i, tab_c, tab_t, idx_i, idx_c, idx_t,
             out_i, out_c, out_t, sl_i, sl_c, sl_t,
             idx_f, rows_v, slen_v, sem):
    wid = lax.axis_index("s") * NC + lax.axis_index("c")
    base = wid * LOOK_W

    for tab, idx_h, out_h, sl_h in ((tab_i, idx_i, out_i, sl_i),
                                    (tab_c, idx_c, out_c, sl_c),
                                    (tab_t, idx_t, out_t, sl_t)):
        pltpu.sync_copy(idx_h.at[pl.ds(base, LOOK_W)], idx_f)

        def chunk_body(c, _, tab=tab, out_h=out_h):
            pltpu.async_copy(
                tab.at[idx_f.at[pl.ds(c * CHUNK, CHUNK)]],
                rows_v, sem).wait()
            pltpu.sync_copy(rows_v, out_h.at[pl.ds(base + c * CHUNK, CHUNK)])
            return _
        lax.fori_loop(0, N_CHUNK, chunk_body, 0)

        acc = [jnp.zeros((LANES,), jnp.int32) for _ in range(VPR)]
        for l in range(L):
            for j in range(VPR):
                v = idx_f[pl.ds(l * ROWS_W + j * LANES, LANES)]
                acc[j] = acc[j] + jnp.where(v != 0, 1, 0).astype(jnp.int32)
        for j in range(VPR):
            slen_v[pl.ds(j * LANES, LANES)] = acc[j]
        pltpu.sync_copy(slen_v, sl_h.at[pl.ds(wid * ROWS_W, ROWS_W)])


@jax.jit
def _run(item_hist, cate_hist, user_tags, table_item, table_cate, table_tags):
    idx1d = lambda a: a.reshape(NW, ROWS_W, L).transpose(0, 2, 1).reshape(-1)

    mesh = plsc.VectorSubcoreMesh(core_axis_name="c", subcore_axis_name="s")
    ewl = jax.ShapeDtypeStruct((B * L, D), jnp.float32)
    f = pl.kernel(
        _sc_body,
        out_type=(ewl, ewl, ewl,
                  jax.ShapeDtypeStruct((B,), jnp.int32),
                  jax.ShapeDtypeStruct((B,), jnp.int32),
                  jax.ShapeDtypeStruct((B,), jnp.int32)),
        mesh=mesh,
        compiler_params=pltpu.CompilerParams(use_tc_tiling_on_sc=False),
        scratch_types=[
            pltpu.VMEM((LOOK_W,), jnp.int32),
            pltpu.VMEM((CHUNK, D), jnp.float32),
            pltpu.VMEM((ROWS_W,), jnp.int32),
            pltpu.SemaphoreType.DMA,
        ],
    )
    lin = lambda t: lax.optimization_barrier(t.reshape(-1)).reshape(t.shape)
    e_i, e_c, e_t, sl_i, sl_c, sl_t = f(
        lin(table_item), lin(table_cate), lin(table_tags),
        idx1d(item_hist), idx1d(cate_hist), idx1d(user_tags),
    )
    unblk = lambda e: (e.reshape(NW, L, ROWS_W, D)
                       .transpose(0, 2, 1, 3).reshape(B, L, D))
    return (unblk(e_i), unblk(e_c), unblk(e_t), sl_i, sl_c, sl_t)


def kernel(item_hist, cate_hist, user_tags, table_item, table_cate, table_tags):
    return _run(item_hist, cate_hist, user_tags,
                table_item, table_cate, table_tags)
